# baseline (speedup 1.0000x reference)
import functools

import jax
import jax.numpy as jnp
from jax import lax
from jax.experimental import pallas as pl
from jax.experimental.pallas import tpu as pltpu
from jax.experimental.pallas import tpu_sc as plsc

EMBED_D = 32
SEQ = 50
BATCH = 4096
B_TOTAL = BATCH * SEQ
NUM_CORES = 2
NUM_SUBCORES = 16
NW = NUM_CORES * NUM_SUBCORES
B_PER_W = B_TOTAL // NW
NB = BATCH // NW
LANES = 16

_mesh = plsc.VectorSubcoreMesh(core_axis_name="c", subcore_axis_name="s")


@functools.partial(
    pl.kernel,
    mesh=_mesh,
    out_type=jax.ShapeDtypeStruct((BATCH, SEQ, EMBED_D), jnp.float32),
    scratch_types=[
        pltpu.VMEM((B_PER_W,), jnp.int32),
        pltpu.SemaphoreType.DMA,
    ],
    compiler_params=pltpu.CompilerParams(
        use_tc_tiling_on_sc=True, needs_layout_passes=False),
)
def _gather_kernel(idx_hbm, tab_hbm, out_hbm, idx_v, sem):
    wid = lax.axis_index("s") * NUM_CORES + lax.axis_index("c")
    base = wid * B_PER_W
    b0 = wid * NB
    pltpu.sync_copy(idx_hbm.at[pl.ds(base, B_PER_W)], idx_v)

    def row_dma(r, b, s):
        pltpu.async_copy(tab_hbm.at[pl.ds(r, 1)],
                         out_hbm.at[b, pl.ds(s, 1)], sem)

    def b_body(bb, _):
        f0 = bb * SEQ
        b = b0 + bb
        vec0 = idx_v[pl.ds(f0, LANES)]
        vec1 = idx_v[pl.ds(f0 + 16, LANES)]
        vec2 = idx_v[pl.ds(f0 + 32, LANES)]
        vec3 = idx_v[pl.ds(f0 + SEQ - LANES, LANES)]
        for l in range(LANES):
            row_dma(vec0[l], b, l)
        for l in range(LANES):
            row_dma(vec1[l], b, 16 + l)
        for l in range(2):
            row_dma(vec2[l], b, 32 + l)
        for l in range(LANES):
            row_dma(vec3[l], b, SEQ - LANES + l)
        return _

    lax.fori_loop(0, NB, b_body, 0)
    pltpu.make_async_copy(tab_hbm.at[pl.ds(0, B_PER_W)],
                          tab_hbm.at[pl.ds(0, B_PER_W)], sem).wait()


def kernel(x, wordmat):
    idx = x.reshape(-1).astype(jnp.int32)
    return _gather_kernel(idx, wordmat)


# speedup vs baseline: 7.7132x; 7.7132x over previous
import functools

import jax
import jax.numpy as jnp
from jax import lax
from jax.experimental import pallas as pl
from jax.experimental.pallas import tpu as pltpu
from jax.experimental.pallas import tpu_sc as plsc

EMBED_D = 32
SEQ = 50
BATCH = 4096
B_TOTAL = BATCH * SEQ
NUM_CORES = 2
NUM_SUBCORES = 16
NW = NUM_CORES * NUM_SUBCORES
B_PER_W = B_TOTAL // NW
NB = BATCH // NW
BCHUNK = 8
N_CHUNKS = NB // BCHUNK
LOOKUPS = BCHUNK * SEQ
LANES = 16

_mesh = plsc.VectorSubcoreMesh(core_axis_name="c", subcore_axis_name="s")


@functools.partial(
    pl.kernel,
    mesh=_mesh,
    out_type=jax.ShapeDtypeStruct((BATCH, SEQ, EMBED_D), jnp.float32),
    scratch_types=[
        pltpu.VMEM((B_PER_W,), jnp.int32),
        pltpu.VMEM((LOOKUPS, EMBED_D), jnp.float32),
        pltpu.VMEM((LOOKUPS, EMBED_D), jnp.float32),
        pltpu.SemaphoreType.DMA,
        pltpu.SemaphoreType.DMA,
        pltpu.SemaphoreType.DMA,
    ],
    compiler_params=pltpu.CompilerParams(
        use_tc_tiling_on_sc=True, needs_layout_passes=False),
)
def _gather_kernel(idx_hbm, tab_hbm, out_hbm, idx_v, buf_a, buf_b,
                   g_sem, wa_sem, wb_sem):
    wid = lax.axis_index("s") * NUM_CORES + lax.axis_index("c")
    base = wid * B_PER_W
    b0 = wid * NB
    pltpu.sync_copy(idx_hbm.at[pl.ds(base, B_PER_W)], idx_v)

    bufs = (buf_a, buf_b)
    wsem = (wa_sem, wb_sem)

    def fill(c, buf):
        def b_body(bb, _):
            f0 = (c * BCHUNK + bb) * SEQ

            def row_dma(r, s):
                pltpu.async_copy(tab_hbm.at[pl.ds(r, 1)],
                                 buf.at[pl.ds(bb * SEQ + s, 1)], g_sem)

            vec0 = idx_v[pl.ds(f0, LANES)]
            vec1 = idx_v[pl.ds(f0 + 16, LANES)]
            vec2 = idx_v[pl.ds(f0 + 32, LANES)]
            vec3 = idx_v[pl.ds(f0 + SEQ - LANES, LANES)]
            for l in range(LANES):
                row_dma(vec0[l], l)
            for l in range(LANES):
                row_dma(vec1[l], 16 + l)
            for l in range(2):
                row_dma(vec2[l], 32 + l)
            for l in range(LANES):
                row_dma(vec3[l], SEQ - LANES + l)
            return _

        lax.fori_loop(0, BCHUNK, b_body, 0)
        pltpu.make_async_copy(tab_hbm.at[pl.ds(0, LOOKUPS)], buf,
                              g_sem).wait()

    def put(c, buf, sem):
        def w_body(bb, _):
            b = b0 + c * BCHUNK + bb
            pltpu.async_copy(buf.at[pl.ds(bb * SEQ, SEQ)], out_hbm.at[b], sem)
            return _
        lax.fori_loop(0, BCHUNK, w_body, 0)

    def drain_put(c, buf, sem):
        def d_body(bb, _):
            pltpu.make_async_copy(buf.at[pl.ds(0, SEQ)], out_hbm.at[b0],
                                  sem).wait()
            return _
        lax.fori_loop(0, BCHUNK, d_body, 0)

    pending = [None, None]
    for c in range(N_CHUNKS):
        p = c % 2
        if pending[p] is not None:
            drain_put(pending[p], bufs[p], wsem[p])
            pending[p] = None
        fill(c, bufs[p])
        put(c, bufs[p], wsem[p])
        pending[p] = c
    for p in range(2):
        if pending[p] is not None:
            drain_put(pending[p], bufs[p], wsem[p])


def kernel(x, wordmat):
    idx = x.reshape(-1).astype(jnp.int32)
    return _gather_kernel(idx, wordmat)
